# baseline (speedup 1.0000x reference)
import jax
import jax.numpy as jnp
import numpy as np
from jax import lax
from jax.experimental import pallas as pl
from jax.experimental.pallas import tpu as pltpu
from jax.experimental.pallas import tpu_sc as plsc

_HEADS = 16
_SIZE = 32
_ROWS = _SIZE * _SIZE
_COLS = _ROWS + 1
_NE = (2 * _SIZE - 1) ** 2
_TBL_PAD = 8192
_G = 32
_RCHUNKS = _ROWS // 16
_GROUPS_PER_SUB = (_ROWS // 2) // _G
_ZSLOT = 4100


def _make_idxb():
  f = np.arange(16 * _ROWS)
  i = f // _ROWS
  c = f % _ROWS
  cd = np.maximum(c - 1, 0)
  t = 63 * (cd >> 5) + (cd & 31)
  idxb = np.where(c == 0, _ZSLOT, i - t + 1984)
  return jnp.asarray(idxb.reshape(16, _ROWS), dtype=jnp.int32)


def _sc_body(tableT_hbm, idxb_hbm, out_hbm, tbl_v, idxb_v, buf0, buf1, sem0, sem1):
  nc = 2
  cid = lax.axis_index("c")
  sid = lax.axis_index("s")
  wid = sid * nc + cid
  head = wid // 2
  half = wid - head * 2
  hi0 = half * _GROUPS_PER_SUB

  pltpu.sync_copy(tableT_hbm.at[head], tbl_v)
  pltpu.sync_copy(idxb_hbm, idxb_v)

  iota = lax.iota(jnp.int32, 16)
  col_last = jnp.full((16,), _ROWS, jnp.int32)

  bufs = (buf0, buf1)
  sems = (sem0, sem1)

  def build_group(hi, buf):
    vdyn = jnp.full((16,), 63 * hi, jnp.int32)

    def row_pair(i, carry):
      @plsc.parallel_loop(0, _RCHUNKS, step=1, unroll=16)
      def _chunk(k):
        off = k * 16
        idx = idxb_v[i, pl.ds(off, 16)] + vdyn
        buf[i, pl.ds(off, 16)] = plsc.load_gather(tbl_v, [idx])
        buf[i + 16, pl.ds(off, 16)] = plsc.load_gather(tbl_v, [idx + 16])
      return carry

    lax.fori_loop(0, 16, row_pair, 0)
    plsc.store_scatter(buf, [iota, col_last],
                       plsc.load_gather(tbl_v, [vdyn + iota]))
    plsc.store_scatter(buf, [iota + 16, col_last],
                       plsc.load_gather(tbl_v, [vdyn + iota + 16]))

  def dma(b, hi):
    return pltpu.make_async_copy(
        bufs[b],
        out_hbm.at[head, pl.ds(hi * _G, _G)],
        sems[b],
    )

  def step(k, carry):
    for b in range(2):
      hi = hi0 + 2 * k + b

      @pl.when(k > 0)
      def _wait():
        dma(b, hi).wait()

      build_group(hi, bufs[b])
      dma(b, hi).start()
    return carry

  lax.fori_loop(0, _GROUPS_PER_SUB // 2, step, 0)
  for b in range(2):
    dma(b, hi0 + b).wait()


@jax.jit
def _rel_pos_bias(pos_bias_table):
  tableT = jnp.zeros((_HEADS, _TBL_PAD), jnp.float32)
  tableT = tableT.at[:, :_NE].set(pos_bias_table.T)
  idxb = _make_idxb()

  mesh = plsc.VectorSubcoreMesh(core_axis_name="c", subcore_axis_name="s")
  call = pl.kernel(
      _sc_body,
      out_type=jax.ShapeDtypeStruct((_HEADS, _ROWS, _COLS), jnp.float32),
      mesh=mesh,
      compiler_params=pltpu.CompilerParams(
          needs_layout_passes=False, use_tc_tiling_on_sc=True),
      scratch_types=[
          pltpu.VMEM((_TBL_PAD,), jnp.float32),
          pltpu.VMEM((16, _ROWS), jnp.int32),
          pltpu.VMEM((_G, _COLS), jnp.float32),
          pltpu.VMEM((_G, _COLS), jnp.float32),
          pltpu.SemaphoreType.DMA,
          pltpu.SemaphoreType.DMA,
      ],
  )
  return call(tableT, idxb)


def kernel(qk, pos_bias_table, pos_indices):
  del qk, pos_indices
  return _rel_pos_bias(pos_bias_table)


# speedup vs baseline: 1.1753x; 1.1753x over previous
import jax
import jax.numpy as jnp
from jax import lax
from jax.experimental import pallas as pl
from jax.experimental.pallas import tpu as pltpu
from jax.experimental.pallas import tpu_sc as plsc

_HEADS = 16
_SIZE = 32
_ROWS = _SIZE * _SIZE
_COLS = _ROWS + 1
_NE = (2 * _SIZE - 1) ** 2
_TBL_PAD = 8192
_WDATA = 1408
_WSTRIDE = 1792
_ZOFF = 1408
_GC = 32
_NG = 16


def _sc_body(tableT_hbm, out_hbm, win_v, buf0, buf1, bufz, sem0, sem1):
  nc = 2
  cid = lax.axis_index("c")
  sid = lax.axis_index("s")
  wid = sid * nc + cid
  ho = wid % 2
  rc = (wid // 2) % 8
  ch = wid // 16

  t_max = 976 + ch * 1008
  aligned_lo = pl.multiple_of(((252 * rc + 1984 - t_max) >> 7) << 7, 128)
  pltpu.sync_copy(
      tableT_hbm.at[pl.ds(ho * 8, 8), pl.ds(aligned_lo, _WDATA)],
      win_v.at[:, pl.ds(0, _WDATA)],
  )

  zeros16 = jnp.zeros((16,), jnp.float32)
  for h in range(8):
    for q in range(_WDATA // 16, _WSTRIDE // 16):
      win_v[h, pl.ds(16 * q, 16)] = zeros16

  iota = lax.iota(jnp.int32, 16)
  sb = []
  for q in range(8):
    rr = iota + 16 * q
    sb.append(((rr >> 5) * 63) + (rr & 31))
  hvec = [jnp.full((16,), h, jnp.int32) for h in range(8)]

  c0 = ch * 512
  bufs = (buf0, buf1)
  sems = (sem0, sem1)

  def build_col(j, buf, gbase):
    c = gbase + j
    cd = jnp.maximum(c - 1, 0)
    t = ((cd >> 5) * 63) + (cd & 31)
    d = jnp.where(c == 0, _ZOFF, 1984 - t - aligned_lo + 252 * rc)
    vd = jnp.full((16,), d, jnp.int32)
    for h in range(8):
      for q in range(8):
        buf[j, h, pl.ds(16 * q, 16)] = plsc.load_gather(
            win_v, [hvec[h], sb[q] + vd])

  def dma(b, g):
    return pltpu.make_async_copy(
        bufs[b],
        out_hbm.at[pl.ds(c0 + g * _GC, _GC), pl.ds(ho * 8, 8),
                   pl.ds(rc * 128, 128)],
        sems[b],
    )

  def step(k, carry):
    for b in range(2):
      g = 2 * k + b

      @pl.when(k > 0)
      def _wait():
        dma(b, g).wait()

      gbase = c0 + g * _GC

      @plsc.parallel_loop(0, _GC, step=1, unroll=2)
      def _col(j):
        build_col(j, bufs[b], gbase)

      dma(b, g).start()
    return carry

  lax.fori_loop(0, _NG // 2, step, 0)
  for b in range(2):
    dma(b, b).wait()

  @pl.when(ch == 1)
  def _last_col():
    vd = jnp.full((16,), 252 * rc - aligned_lo, jnp.int32)
    for h in range(8):
      for q in range(8):
        bufz[0, h, pl.ds(16 * q, 16)] = plsc.load_gather(
            win_v, [hvec[h], sb[q] + vd])
    pltpu.sync_copy(
        bufz,
        out_hbm.at[pl.ds(_COLS - 1, 1), pl.ds(ho * 8, 8), pl.ds(rc * 128, 128)],
    )


@jax.jit
def _rel_pos_bias(pos_bias_table):
  tableT = jnp.zeros((_HEADS, _TBL_PAD), jnp.float32)
  tableT = tableT.at[:, :_NE].set(pos_bias_table.T)

  mesh = plsc.VectorSubcoreMesh(core_axis_name="c", subcore_axis_name="s")
  call = pl.kernel(
      _sc_body,
      out_type=jax.ShapeDtypeStruct((_COLS, _HEADS, _ROWS), jnp.float32),
      mesh=mesh,
      compiler_params=pltpu.CompilerParams(needs_layout_passes=False),
      scratch_types=[
          pltpu.VMEM((8, _WSTRIDE), jnp.float32),
          pltpu.VMEM((_GC, 8, 128), jnp.float32),
          pltpu.VMEM((_GC, 8, 128), jnp.float32),
          pltpu.VMEM((1, 8, 128), jnp.float32),
          pltpu.SemaphoreType.DMA,
          pltpu.SemaphoreType.DMA,
      ],
  )
  out = call(tableT)
  return jnp.transpose(out, (1, 2, 0))


def kernel(qk, pos_bias_table, pos_indices):
  del qk, pos_indices
  return _rel_pos_bias(pos_bias_table)


# speedup vs baseline: 1.9923x; 1.6952x over previous
import jax
import jax.numpy as jnp
from jax import lax
from jax.experimental import pallas as pl
from jax.experimental.pallas import tpu as pltpu
from jax.experimental.pallas import tpu_sc as plsc

_HEADS = 16
_SIZE = 32
_ROWS = _SIZE * _SIZE
_COLS = _ROWS + 1
_NE = (2 * _SIZE - 1) ** 2
_TBL_PAD = 8192
_WDATA = 1408
_WSTRIDE = 1792
_ZOFF = 1408
_GC = 32
_NG = 16


def _sc_body(tableT_hbm, out_hbm, win_v, buf0, buf1, bufz, sem0, sem1):
  nc = 2
  cid = lax.axis_index("c")
  sid = lax.axis_index("s")
  wid = sid * nc + cid
  ho = wid % 2
  rc = (wid // 2) % 8
  ch = wid // 16

  t_max = 976 + ch * 1008
  aligned_lo = pl.multiple_of(((252 * rc + 1984 - t_max) >> 7) << 7, 128)
  pltpu.sync_copy(
      tableT_hbm.at[pl.ds(ho * 8, 8), pl.ds(aligned_lo, _WDATA)],
      win_v.at[:, pl.ds(0, _WDATA)],
  )

  zeros16 = jnp.zeros((16,), jnp.float32)
  for h in range(8):
    for q in range(_WDATA // 16, _WSTRIDE // 16):
      win_v[h, pl.ds(16 * q, 16)] = zeros16

  iota = lax.iota(jnp.int32, 16)
  sb = []
  for q in range(8):
    rr = iota + 16 * q
    sb.append(((rr >> 5) * 63) + (rr & 31))
  hvec = [jnp.full((16,), h, jnp.int32) for h in range(8)]

  c0 = ch * 512
  bufs = (buf0, buf1)
  sems = (sem0, sem1)

  def build_col(j, buf, gbase):
    c = gbase + j
    cd = jnp.maximum(c - 1, 0)
    t = ((cd >> 5) * 63) + (cd & 31)
    d = jnp.where(c == 0, _ZOFF, 1984 - t - aligned_lo + 252 * rc)
    vd = jnp.full((16,), d, jnp.int32)
    for h in range(8):
      for q in range(8):
        buf[j, h, pl.ds(16 * q, 16)] = plsc.load_gather(
            win_v, [hvec[h], sb[q] + vd])

  def dma(b, g):
    return pltpu.make_async_copy(
        bufs[b],
        out_hbm.at[pl.ds(c0 + g * _GC, _GC), pl.ds(ho * 8, 8),
                   pl.ds(rc * 128, 128)],
        sems[b],
    )

  def step(k, carry):
    for b in range(2):
      g = 2 * k + b

      @pl.when(k > 0)
      def _wait():
        dma(b, g).wait()

      gbase = c0 + g * _GC

      @plsc.parallel_loop(0, _GC, step=1, unroll=1)
      def _col(j):
        build_col(j, bufs[b], gbase)

      dma(b, g).start()
    return carry

  lax.fori_loop(0, _NG // 2, step, 0)
  for b in range(2):
    dma(b, b).wait()

  @pl.when(ch == 1)
  def _last_col():
    vd = jnp.full((16,), 252 * rc - aligned_lo, jnp.int32)
    for h in range(8):
      for q in range(8):
        bufz[0, h, pl.ds(16 * q, 16)] = plsc.load_gather(
            win_v, [hvec[h], sb[q] + vd])
    pltpu.sync_copy(
        bufz,
        out_hbm.at[pl.ds(_COLS - 1, 1), pl.ds(ho * 8, 8), pl.ds(rc * 128, 128)],
    )


@jax.jit
def _rel_pos_bias(pos_bias_table):
  tableT = jnp.zeros((_HEADS, _TBL_PAD), jnp.float32)
  tableT = tableT.at[:, :_NE].set(pos_bias_table.T)

  mesh = plsc.VectorSubcoreMesh(core_axis_name="c", subcore_axis_name="s")
  call = pl.kernel(
      _sc_body,
      out_type=jax.ShapeDtypeStruct((_COLS, _HEADS, _ROWS), jnp.float32),
      mesh=mesh,
      compiler_params=pltpu.CompilerParams(needs_layout_passes=False),
      scratch_types=[
          pltpu.VMEM((8, _WSTRIDE), jnp.float32),
          pltpu.VMEM((_GC, 8, 128), jnp.float32),
          pltpu.VMEM((_GC, 8, 128), jnp.float32),
          pltpu.VMEM((1, 8, 128), jnp.float32),
          pltpu.SemaphoreType.DMA,
          pltpu.SemaphoreType.DMA,
      ],
  )
  out = call(tableT)
  return jnp.transpose(out, (1, 2, 0))


def kernel(qk, pos_bias_table, pos_indices):
  del qk, pos_indices
  return _rel_pos_bias(pos_bias_table)
